# baseline (speedup 1.0000x reference)
import functools

import jax
import jax.numpy as jnp
from jax.experimental import pallas as pl
from jax.experimental.pallas import tpu as pltpu

_SIZES = [256, 256, 128, 256, 256]
_NL = 4
_B = 1024
_BB = 128
_C = 8


def _prep_body(*refs):
    ins = refs[:3 * _NL]
    outs = refs[3 * _NL:]
    for l in range(_NL):
        a0 = ins[3 * l][...]
        a1 = ins[3 * l + 1][...]
        opsT = ins[3 * l + 2][...]
        m = a1 > a0
        f = (opsT == 0).reshape(1, -1)
        sign = jnp.where(f, -1.0, 1.0)
        outs[2 * l][...] = jnp.where(m, sign, 0.0).astype(jnp.float32)
        outs[2 * l + 1][...] = jnp.where(m & f, 1.0, 0.0).astype(jnp.float32)


def _fwd_body(*refs):
    pq = refs[:2 * _NL]
    opsT = refs[2 * _NL:3 * _NL]
    xT_ref = refs[3 * _NL]
    out_ref = refs[3 * _NL + 1]

    x = xT_ref[...]
    for l in range(_NL):
        fin = _SIZES[l]
        fout = _SIZES[l + 1]
        PT = pq[2 * l]
        QT = pq[2 * l + 1]
        acc = jnp.zeros((fout, _BB), dtype=jnp.float32)
        for s in range(fin // _C):
            sl = slice(s * _C, (s + 1) * _C)
            ev = (PT[sl][:, :, None] * x[sl][:, None, :]
                  + QT[sl][:, :, None])
            acc = jnp.maximum(acc, jnp.max(ev, axis=0))
        f = opsT[l][...] == 0
        x = jnp.where(f, 1.0 - acc, acc)
    out_ref[...] = x


@functools.partial(jax.jit, static_argnums=())
def kernel(x, logits_0, logits_1, logits_2, logits_3,
           ops_0, ops_1, ops_2, ops_3,
           gnoise_0, gnoise_1, gnoise_2, gnoise_3):
    logits = [logits_0, logits_1, logits_2, logits_3]
    gnoise = [gnoise_0, gnoise_1, gnoise_2, gnoise_3]
    ops = [ops_0, ops_1, ops_2, ops_3]

    prep_in = []
    prep_in_specs = []
    for l in range(_NL):
        a = logits[l] + gnoise[l]
        prep_in.append(a[:, :, 0].T)
        prep_in.append(a[:, :, 1].T)
        prep_in.append(ops[l].reshape(-1, 1))
        fin, fout = _SIZES[l], _SIZES[l + 1]
        prep_in_specs += [
            pl.BlockSpec((fin, fout), lambda: (0, 0)),
            pl.BlockSpec((fin, fout), lambda: (0, 0)),
            pl.BlockSpec((fout, 1), lambda: (0, 0)),
        ]

    pq_types = []
    for l in range(_NL):
        fin, fout = _SIZES[l], _SIZES[l + 1]
        pq_types += [jax.ShapeDtypeStruct((fin, fout), jnp.float32)] * 2

    pq = pl.pallas_call(
        _prep_body,
        out_shape=tuple(pq_types),
    )(*prep_in)

    xT = x.T

    fwd_in = list(pq) + [ops[l].reshape(-1, 1) for l in range(_NL)] + [xT]
    fwd_specs = []
    for l in range(_NL):
        fin, fout = _SIZES[l], _SIZES[l + 1]
        fwd_specs += [
            pl.BlockSpec((fin, fout), lambda j: (0, 0)),
            pl.BlockSpec((fin, fout), lambda j: (0, 0)),
        ]
    for l in range(_NL):
        fout = _SIZES[l + 1]
        fwd_specs.append(pl.BlockSpec((fout, 1), lambda j: (0, 0)))
    fwd_specs.append(pl.BlockSpec((_SIZES[0], _BB), lambda j: (0, j)))

    yT = pl.pallas_call(
        _fwd_body,
        grid=(_B // _BB,),
        in_specs=fwd_specs,
        out_specs=pl.BlockSpec((_SIZES[_NL], _BB), lambda j: (0, j)),
        out_shape=jax.ShapeDtypeStruct((_SIZES[_NL], _B), jnp.float32),
        compiler_params=pltpu.CompilerParams(
            dimension_semantics=("parallel",),
        ),
    )(*fwd_in)

    return yT.T


# speedup vs baseline: 3.4382x; 3.4382x over previous
import jax
import jax.numpy as jnp
from jax.experimental import pallas as pl
from jax.experimental.pallas import tpu as pltpu

_SIZES = [256, 256, 128, 256, 256]
_NL = 4
_B = 1024
_BB = 128


def _prep_body(*refs):
    ins = refs[:3 * _NL]
    outs = refs[3 * _NL:]
    for l in range(_NL):
        a0 = ins[3 * l][...]
        a1 = ins[3 * l + 1][...]
        ops_row = ins[3 * l + 2][...]
        m = a1 > a0
        f = ops_row == 0
        sign = jnp.where(f, -1.0, 1.0)
        outs[2 * l][...] = jnp.where(m, sign, 0.0).astype(jnp.float32)
        outs[2 * l + 1][...] = jnp.where(m & f, 1.0, 0.0).astype(jnp.float32)


def _fwd_body(*refs):
    pq = refs[:2 * _NL]
    ops_rows = refs[2 * _NL:3 * _NL]
    x_ref = refs[3 * _NL]
    out_ref = refs[3 * _NL + 1]

    x = x_ref[...]
    for l in range(_NL):
        fin = _SIZES[l]
        fout = _SIZES[l + 1]
        PT = pq[2 * l][...]
        QT = pq[2 * l + 1][...]
        acc = jnp.zeros((_BB, fout), dtype=jnp.float32)
        for i in range(fin):
            acc = jnp.maximum(acc, x[:, i:i + 1] * PT[i:i + 1, :] + QT[i:i + 1, :])
        f = ops_rows[l][...] == 0
        x = jnp.where(f, 1.0 - acc, acc)
    out_ref[...] = x


def kernel(x, logits_0, logits_1, logits_2, logits_3,
           ops_0, ops_1, ops_2, ops_3,
           gnoise_0, gnoise_1, gnoise_2, gnoise_3):
    logits = [logits_0, logits_1, logits_2, logits_3]
    gnoise = [gnoise_0, gnoise_1, gnoise_2, gnoise_3]
    ops = [ops_0, ops_1, ops_2, ops_3]

    prep_in = []
    for l in range(_NL):
        a = logits[l] + gnoise[l]
        prep_in.append(a[:, :, 0].T)
        prep_in.append(a[:, :, 1].T)
        prep_in.append(ops[l].reshape(1, -1))

    pq_types = []
    for l in range(_NL):
        fin, fout = _SIZES[l], _SIZES[l + 1]
        pq_types += [jax.ShapeDtypeStruct((fin, fout), jnp.float32)] * 2

    pq = pl.pallas_call(
        _prep_body,
        out_shape=tuple(pq_types),
    )(*prep_in)

    fwd_in = list(pq) + [ops[l].reshape(1, -1) for l in range(_NL)] + [x]
    fwd_specs = []
    for l in range(_NL):
        fin, fout = _SIZES[l], _SIZES[l + 1]
        fwd_specs += [
            pl.BlockSpec((fin, fout), lambda j: (0, 0)),
            pl.BlockSpec((fin, fout), lambda j: (0, 0)),
        ]
    for l in range(_NL):
        fout = _SIZES[l + 1]
        fwd_specs.append(pl.BlockSpec((1, fout), lambda j: (0, 0)))
    fwd_specs.append(pl.BlockSpec((_BB, _SIZES[0]), lambda j: (j, 0)))

    y = pl.pallas_call(
        _fwd_body,
        grid=(_B // _BB,),
        in_specs=fwd_specs,
        out_specs=pl.BlockSpec((_BB, _SIZES[_NL]), lambda j: (j, 0)),
        out_shape=jax.ShapeDtypeStruct((_B, _SIZES[_NL]), jnp.float32),
        compiler_params=pltpu.CompilerParams(
            dimension_semantics=("parallel",),
        ),
    )(*fwd_in)

    return y


# speedup vs baseline: 5.9266x; 1.7238x over previous
import jax
import jax.numpy as jnp
from jax.experimental import pallas as pl
from jax.experimental.pallas import tpu as pltpu

_SIZES = [256, 256, 128, 256, 256]
_NL = 4
_B = 1024
_BB = 128


def _prep_body(*refs):
    ins = refs[:3 * _NL]
    outs = refs[3 * _NL:]
    for l in range(_NL):
        a0 = ins[3 * l][...]
        a1 = ins[3 * l + 1][...]
        ops_row = ins[3 * l + 2][...]
        m = a1 > a0
        f = ops_row == 0
        sign = jnp.where(f, -1.0, 1.0)
        outs[2 * l][...] = jnp.where(m, sign, 0.0).astype(jnp.bfloat16)
        outs[2 * l + 1][...] = jnp.where(m & f, 1.0, 0.0).astype(jnp.bfloat16)


def _fwd_body(*refs):
    pq = refs[:2 * _NL]
    ops_rows = refs[2 * _NL:3 * _NL]
    x_ref = refs[3 * _NL]
    out_ref = refs[3 * _NL + 1]

    x = x_ref[...].astype(jnp.bfloat16)
    for l in range(_NL):
        fin = _SIZES[l]
        fout = _SIZES[l + 1]
        PT = pq[2 * l][...]
        QT = pq[2 * l + 1][...]
        acc = jnp.zeros((_BB, fout), dtype=jnp.bfloat16)
        for i in range(fin):
            acc = jnp.maximum(acc, x[:, i:i + 1] * PT[i:i + 1, :] + QT[i:i + 1, :])
        f = ops_rows[l][...] == 0
        x = jnp.where(f, jnp.bfloat16(1.0) - acc, acc)
    out_ref[...] = x.astype(jnp.float32)


def kernel(x, logits_0, logits_1, logits_2, logits_3,
           ops_0, ops_1, ops_2, ops_3,
           gnoise_0, gnoise_1, gnoise_2, gnoise_3):
    logits = [logits_0, logits_1, logits_2, logits_3]
    gnoise = [gnoise_0, gnoise_1, gnoise_2, gnoise_3]
    ops = [ops_0, ops_1, ops_2, ops_3]

    prep_in = []
    for l in range(_NL):
        a = logits[l] + gnoise[l]
        prep_in.append(a[:, :, 0].T)
        prep_in.append(a[:, :, 1].T)
        prep_in.append(ops[l].reshape(1, -1))

    pq_types = []
    for l in range(_NL):
        fin, fout = _SIZES[l], _SIZES[l + 1]
        pq_types += [jax.ShapeDtypeStruct((fin, fout), jnp.bfloat16)] * 2

    pq = pl.pallas_call(
        _prep_body,
        out_shape=tuple(pq_types),
    )(*prep_in)

    fwd_in = list(pq) + [ops[l].reshape(1, -1) for l in range(_NL)] + [x]
    fwd_specs = []
    for l in range(_NL):
        fin, fout = _SIZES[l], _SIZES[l + 1]
        fwd_specs += [
            pl.BlockSpec((fin, fout), lambda j: (0, 0)),
            pl.BlockSpec((fin, fout), lambda j: (0, 0)),
        ]
    for l in range(_NL):
        fout = _SIZES[l + 1]
        fwd_specs.append(pl.BlockSpec((1, fout), lambda j: (0, 0)))
    fwd_specs.append(pl.BlockSpec((_BB, _SIZES[0]), lambda j: (j, 0)))

    y = pl.pallas_call(
        _fwd_body,
        grid=(_B // _BB,),
        in_specs=fwd_specs,
        out_specs=pl.BlockSpec((_BB, _SIZES[_NL]), lambda j: (j, 0)),
        out_shape=jax.ShapeDtypeStruct((_B, _SIZES[_NL]), jnp.float32),
        compiler_params=pltpu.CompilerParams(
            dimension_semantics=("parallel",),
        ),
    )(*fwd_in)

    return y


# speedup vs baseline: 6.0626x; 1.0229x over previous
import jax
import jax.numpy as jnp
from jax.experimental import pallas as pl
from jax.experimental.pallas import tpu as pltpu

_SIZES = [256, 256, 128, 256, 256]
_NL = 4
_B = 1024
_BB = 128


def _fwd_body(*refs):
    ins = refs[:3 * _NL]
    x_ref = refs[3 * _NL]
    out_ref = refs[3 * _NL + 1]
    pq = refs[3 * _NL + 2:]

    @pl.when(pl.program_id(0) == 0)
    def _prep():
        for l in range(_NL):
            a0 = ins[3 * l][...]
            a1 = ins[3 * l + 1][...]
            ops_row = ins[3 * l + 2][...]
            m = a1 > a0
            f = ops_row == 0
            sign = jnp.where(f, -1.0, 1.0)
            pq[2 * l][...] = jnp.where(m, sign, 0.0).astype(jnp.bfloat16)
            pq[2 * l + 1][...] = jnp.where(m & f, 1.0, 0.0).astype(jnp.bfloat16)

    x = x_ref[...].astype(jnp.bfloat16)
    for l in range(_NL):
        fin = _SIZES[l]
        fout = _SIZES[l + 1]
        PT = pq[2 * l][...]
        QT = pq[2 * l + 1][...]
        acc = jnp.zeros((_BB, fout), dtype=jnp.bfloat16)
        for i in range(fin):
            acc = jnp.maximum(acc, x[:, i:i + 1] * PT[i:i + 1, :] + QT[i:i + 1, :])
        f = ins[3 * l + 2][...] == 0
        x = jnp.where(f, jnp.bfloat16(1.0) - acc, acc)
    out_ref[...] = x.astype(jnp.float32)


def kernel(x, logits_0, logits_1, logits_2, logits_3,
           ops_0, ops_1, ops_2, ops_3,
           gnoise_0, gnoise_1, gnoise_2, gnoise_3):
    logits = [logits_0, logits_1, logits_2, logits_3]
    gnoise = [gnoise_0, gnoise_1, gnoise_2, gnoise_3]
    ops = [ops_0, ops_1, ops_2, ops_3]

    fwd_in = []
    fwd_specs = []
    for l in range(_NL):
        a = logits[l] + gnoise[l]
        fwd_in.append(a[:, :, 0].T)
        fwd_in.append(a[:, :, 1].T)
        fwd_in.append(ops[l].reshape(1, -1))
        fin, fout = _SIZES[l], _SIZES[l + 1]
        fwd_specs += [
            pl.BlockSpec((fin, fout), lambda j: (0, 0)),
            pl.BlockSpec((fin, fout), lambda j: (0, 0)),
            pl.BlockSpec((1, fout), lambda j: (0, 0)),
        ]
    fwd_in.append(x)
    fwd_specs.append(pl.BlockSpec((_BB, _SIZES[0]), lambda j: (j, 0)))

    scratch = []
    for l in range(_NL):
        fin, fout = _SIZES[l], _SIZES[l + 1]
        scratch += [pltpu.VMEM((fin, fout), jnp.bfloat16)] * 2

    y = pl.pallas_call(
        _fwd_body,
        grid=(_B // _BB,),
        in_specs=fwd_specs,
        out_specs=pl.BlockSpec((_BB, _SIZES[_NL]), lambda j: (j, 0)),
        out_shape=jax.ShapeDtypeStruct((_B, _SIZES[_NL]), jnp.float32),
        scratch_shapes=scratch,
        compiler_params=pltpu.CompilerParams(
            dimension_semantics=("arbitrary",),
        ),
    )(*fwd_in)

    return y
